# baseline (speedup 1.0000x reference)
import functools

import jax
import jax.numpy as jnp
from jax import lax
from jax.experimental import pallas as pl
from jax.experimental.pallas import tpu as pltpu
from jax.experimental.pallas import tpu_sc as plsc

_D = 128
_B = 16384
_EPS = 1e-5

_NC = 2
_NS = 16
_NW = _NC * _NS
_B_PER_W = _B // _NW


def _dense_body(emb_ref, w_ref, b_ref, g_ref, beta_ref, out_ref):
    emb = emb_ref[...]
    w = w_ref[...]
    h = lax.dot_general(emb, w, (((1,), (1,)), ((), ())),
                        preferred_element_type=jnp.float32)
    h = h + b_ref[...]
    mu = jnp.mean(h, axis=-1, keepdims=True)
    d = h - mu
    var = jnp.mean(d * d, axis=-1, keepdims=True)
    out_ref[...] = d * lax.rsqrt(var + _EPS) * g_ref[...] + beta_ref[...]


_dense_call = pl.pallas_call(
    _dense_body,
    out_shape=jax.ShapeDtypeStruct((8, _D), jnp.float32),
)


def _gather_body(table_hbm, idx_hbm, out_hbm, idx_v, rows_v, sem):
    wid = lax.axis_index("s") * _NC + lax.axis_index("c")
    base = wid * _B_PER_W
    pltpu.sync_copy(idx_hbm.at[pl.ds(base, _B_PER_W)], idx_v)
    pltpu.async_copy(table_hbm.at[idx_v], rows_v, sem).wait()
    pltpu.sync_copy(rows_v, out_hbm.at[pl.ds(base, _B_PER_W)])


@functools.cache
def _gather_call():
    return pl.kernel(
        _gather_body,
        out_type=jax.ShapeDtypeStruct((_B, _D), jnp.float32),
        mesh=plsc.VectorSubcoreMesh(core_axis_name="c", subcore_axis_name="s",
                                    num_cores=_NC, num_subcores=_NS),
        scratch_types=[
            pltpu.VMEM((_B_PER_W,), jnp.int32),
            pltpu.VMEM((_B_PER_W, _D), jnp.float32),
            pltpu.SemaphoreType.DMA,
        ],
    )


def kernel(x, emb_table, W, b, gamma, beta):
    idx = x.astype(jnp.int32)
    emb_pad = jnp.pad(emb_table, ((0, 8 - emb_table.shape[0]), (0, 0)))
    table = _dense_call(emb_pad, W, b.reshape(1, _D),
                        gamma.reshape(1, _D), beta.reshape(1, _D))
    return _gather_call()(table, idx)


# speedup vs baseline: 12.6692x; 12.6692x over previous
import functools

import jax
import jax.numpy as jnp
from jax import lax
from jax.experimental import pallas as pl
from jax.experimental.pallas import tpu as pltpu
from jax.experimental.pallas import tpu_sc as plsc

_D = 128
_B = 16384
_EPS = 1e-5

_NC = 2
_NS = 16
_NW = _NC * _NS
_B_PER_W = _B // _NW


def _dense_body(emb_ref, w_ref, b_ref, g_ref, beta_ref, out_ref):
    emb = emb_ref[...]
    w = w_ref[...]
    h = lax.dot_general(emb, w, (((1,), (1,)), ((), ())),
                        preferred_element_type=jnp.float32)
    h = h + b_ref[...]
    mu = jnp.mean(h, axis=-1, keepdims=True)
    d = h - mu
    var = jnp.mean(d * d, axis=-1, keepdims=True)
    out_ref[...] = d * lax.rsqrt(var + _EPS) * g_ref[...] + beta_ref[...]


_dense_call = pl.pallas_call(
    _dense_body,
    out_shape=jax.ShapeDtypeStruct((8, _D), jnp.float32),
)


_L = 16
_NCG = _D // _L


def _gather_body(table_hbm, idx_hbm, out_hbm, table_v, idx_v, rows_v, sem):
    wid = lax.axis_index("s") * _NC + lax.axis_index("c")
    base = wid * _B_PER_W
    pltpu.sync_copy(table_hbm.at[pl.ds(0, 8)], table_v)
    pltpu.sync_copy(idx_hbm.at[pl.ds(base, _B_PER_W)], idx_v)
    r0 = [table_v[0, pl.ds(_L * c, _L)] for c in range(_NCG)]
    dl = [table_v[1, pl.ds(_L * c, _L)] - r0[c] for c in range(_NCG)]

    def body(j, carry):
        xv = idx_v[pl.ds(j * _L, _L)].astype(jnp.float32)
        for l in range(_L):
            xf = jnp.full((_L,), xv[l], jnp.float32)
            i = j * _L + l
            for c in range(_NCG):
                rows_v[i, pl.ds(_L * c, _L)] = r0[c] + xf * dl[c]
        return carry

    lax.fori_loop(0, _B_PER_W // _L, body, 0)
    pltpu.sync_copy(rows_v, out_hbm.at[pl.ds(base, _B_PER_W)])


@functools.cache
def _gather_call():
    return pl.kernel(
        _gather_body,
        out_type=jax.ShapeDtypeStruct((_B, _D), jnp.float32),
        mesh=plsc.VectorSubcoreMesh(core_axis_name="c", subcore_axis_name="s",
                                    num_cores=_NC, num_subcores=_NS),
        scratch_types=[
            pltpu.VMEM((8, _D), jnp.float32),
            pltpu.VMEM((_B_PER_W,), jnp.int32),
            pltpu.VMEM((_B_PER_W, _D), jnp.float32),
            pltpu.SemaphoreType.DMA,
        ],
    )


def kernel(x, emb_table, W, b, gamma, beta):
    idx = x.astype(jnp.int32)
    emb_pad = jnp.pad(emb_table, ((0, 8 - emb_table.shape[0]), (0, 0)))
    table = _dense_call(emb_pad, W, b.reshape(1, _D),
                        gamma.reshape(1, _D), beta.reshape(1, _D))
    return _gather_call()(table, idx)
